# baseline (speedup 1.0000x reference)
import functools

import jax
import jax.numpy as jnp
from jax import lax
from jax.experimental import pallas as pl
from jax.experimental.pallas import tpu as pltpu
from jax.experimental.pallas import tpu_sc as plsc

B = 128
S = 512
H = 768
L = 16
NC = 2
NS = 16
NW = NC * NS
P = S // NW
HL = H // L


def _rsqrt(x_v):
    i = lax.bitcast_convert_type(x_v, jnp.int32)
    i = jnp.full((L,), 0x5F3759DF, jnp.int32) - lax.shift_right_logical(
        i, jnp.full((L,), 1, jnp.int32))
    y = lax.bitcast_convert_type(i, jnp.float32)
    half = x_v * 0.5
    for _ in range(4):
        y = y * (1.5 - half * y * y)
    return y


def _sc_body(ids_hbm, tt_hbm, word_hbm, pos_hbm, type_hbm, gamma_hbm,
             beta_hbm, out_hbm, ids_row, tt_row, combo_v, gam_v, bet_v,
             wbuf, sem):
    wid = lax.axis_index("s") * NC + lax.axis_index("c")
    p0 = wid * P

    pltpu.sync_copy(gamma_hbm, gam_v)
    pltpu.sync_copy(beta_hbm, bet_v)
    pltpu.sync_copy(pos_hbm.at[pl.ds(p0, P)], combo_v.at[0])
    pltpu.sync_copy(pos_hbm.at[pl.ds(p0, P)], combo_v.at[1])
    pltpu.sync_copy(type_hbm, wbuf.at[pl.ds(0, 2)])

    def combo_body(i, _):
        ty = i // (P * HL)
        r = i % (P * HL)
        j = r // HL
        h = r % HL
        combo_v[ty, j, pl.ds(h * L, L)] = (
            combo_v[ty, j, pl.ds(h * L, L)] + wbuf[ty, pl.ds(h * L, L)])
        return 0
    lax.fori_loop(0, 2 * P * HL, combo_body, 0)

    def b_body(b, _):
        pltpu.sync_copy(ids_hbm.at[pl.ds(b * S + p0, P)], ids_row)
        pltpu.sync_copy(tt_hbm.at[pl.ds(b * S + p0, P)], tt_row)
        idx = ids_row[:]
        pltpu.async_copy(word_hbm.at[idx], wbuf, sem).wait()
        tt_vec = tt_row[:]
        lanes = lax.iota(jnp.int32, L)

        def j_body(j, _):
            ty = jnp.max(jnp.where(lanes == j, tt_vec, 0))

            zero = jnp.zeros((L,), jnp.float32)

            def h1(h, carry):
                acc, acc2 = carry
                v = wbuf[j, pl.ds(h * L, L)] + combo_v[ty, j, pl.ds(h * L, L)]
                wbuf[j, pl.ds(h * L, L)] = v
                return acc + v, acc2 + v * v
            acc, acc2 = lax.fori_loop(0, HL, h1, (zero, zero))

            s1 = jnp.sum(acc)
            s2 = jnp.sum(acc2)
            mean = s1 * (1.0 / H)
            var = s2 * (1.0 / H) - mean * mean
            inv = _rsqrt(jnp.full((L,), var + 1e-12, jnp.float32))
            mean_v = jnp.full((L,), mean, jnp.float32)

            def h2(h, _):
                v = wbuf[j, pl.ds(h * L, L)]
                o = ((v - mean_v) * inv) * gam_v[pl.ds(h * L, L)] \
                    + bet_v[pl.ds(h * L, L)]
                wbuf[j, pl.ds(h * L, L)] = o
                return 0
            lax.fori_loop(0, HL, h2, 0)
            return 0
        lax.fori_loop(0, P, j_body, 0)

        pltpu.sync_copy(wbuf, out_hbm.at[b, pl.ds(p0, P)])
        return 0
    lax.fori_loop(0, B, b_body, 0)


@functools.partial(jax.jit, static_argnames=())
def kernel(input_ids, token_type_ids, word_emb, pos_emb, type_emb, gamma,
           beta):
    mesh = plsc.VectorSubcoreMesh(
        core_axis_name="c", subcore_axis_name="s",
        num_cores=NC, num_subcores=NS)
    fn = pl.kernel(
        _sc_body,
        out_type=jax.ShapeDtypeStruct((B, S, H), jnp.float32),
        mesh=mesh,
        compiler_params=pltpu.CompilerParams(needs_layout_passes=False),
        scratch_types=[
            pltpu.VMEM((P,), jnp.int32),
            pltpu.VMEM((P,), jnp.int32),
            pltpu.VMEM((2, P, H), jnp.float32),
            pltpu.VMEM((H,), jnp.float32),
            pltpu.VMEM((H,), jnp.float32),
            pltpu.VMEM((P, H), jnp.float32),
            pltpu.SemaphoreType.DMA,
        ],
    )
    return fn(input_ids.reshape(B * S), token_type_ids.reshape(B * S),
              word_emb, pos_emb, type_emb, gamma, beta)


# speedup vs baseline: 4.9252x; 4.9252x over previous
import functools

import jax
import jax.numpy as jnp
from jax import lax
from jax.experimental import pallas as pl
from jax.experimental.pallas import tpu as pltpu
from jax.experimental.pallas import tpu_sc as plsc

B = 128
S = 512
H = 768
L = 16
NC = 2
NS = 16
NW = NC * NS
P = S // NW
HL = H // L


def _rsqrt(x_v):
    i = lax.bitcast_convert_type(x_v, jnp.int32)
    i = jnp.full((L,), 0x5F3759DF, jnp.int32) - lax.shift_right_logical(
        i, jnp.full((L,), 1, jnp.int32))
    y = lax.bitcast_convert_type(i, jnp.float32)
    half = x_v * 0.5
    for _ in range(4):
        y = y * (1.5 - half * y * y)
    return y


def _sc_body(ids_hbm, tt_hbm, word_hbm, pos_hbm, type_hbm, gamma_hbm,
             beta_hbm, out_hbm, ids_v, tt_v, combo_v, gam_v, bet_v,
             wbuf, obuf, in_sem, out_sem, ids_sem, tts_sem):
    wid = lax.axis_index("s") * NC + lax.axis_index("c")
    p0 = wid * P

    pltpu.sync_copy(gamma_hbm, gam_v)
    pltpu.sync_copy(beta_hbm, bet_v)
    pltpu.sync_copy(pos_hbm.at[pl.ds(p0, P)], combo_v.at[0])
    pltpu.sync_copy(pos_hbm.at[pl.ds(p0, P)], combo_v.at[1])
    pltpu.sync_copy(type_hbm, obuf.at[0, pl.ds(0, 2)])

    @plsc.parallel_loop(0, 2 * P * HL, step=1, unroll=4)
    def combo_body(i):
        ty = i // (P * HL)
        r = i % (P * HL)
        j = r // HL
        h = r % HL
        combo_v[ty, j, pl.ds(h * L, L)] = (
            combo_v[ty, j, pl.ds(h * L, L)] + obuf[0, ty, pl.ds(h * L, L)])

    lanes = lax.iota(jnp.int32, L)

    pltpu.sync_copy(ids_hbm.at[pl.ds(p0, P)], ids_v.at[0])
    pltpu.sync_copy(tt_hbm.at[pl.ds(p0, P)], tt_v.at[0])
    pltpu.async_copy(word_hbm.at[ids_v[0, :]], wbuf.at[0], in_sem.at[0])
    pltpu.async_copy(ids_hbm.at[pl.ds(S + p0, P)], ids_v.at[1],
                     ids_sem.at[1])
    pltpu.async_copy(tt_hbm.at[pl.ds(S + p0, P)], tt_v.at[1],
                     tts_sem.at[1])

    def b_body(b, _):
        slot = lax.rem(b, 2)
        nslot = 1 - slot

        @pl.when(b + 1 < B)
        def _():
            pltpu.make_async_copy(
                ids_hbm.at[pl.ds(p0, P)], ids_v.at[nslot],
                ids_sem.at[nslot]).wait()
            pltpu.make_async_copy(
                tt_hbm.at[pl.ds(p0, P)], tt_v.at[nslot],
                tts_sem.at[nslot]).wait()
            pltpu.async_copy(word_hbm.at[ids_v[nslot, :]], wbuf.at[nslot],
                             in_sem.at[nslot])

        pltpu.make_async_copy(
            pos_hbm.at[pl.ds(0, P)], wbuf.at[slot], in_sem.at[slot]).wait()

        @pl.when(b >= 2)
        def _():
            pltpu.make_async_copy(
                obuf.at[slot], out_hbm.at[b, pl.ds(p0, P)],
                out_sem.at[slot]).wait()

        tt_vec = tt_v[slot, :]

        def j_body(j, _):
            ty = jnp.max(jnp.where(lanes == j, tt_vec, 0))

            zero = jnp.zeros((L,), jnp.float32)

            @plsc.parallel_loop(0, HL, step=4, unroll=2,
                                carry=(zero,) * 8)
            def sums(h, c):
                accs = list(c)
                for k in range(4):
                    sl = pl.ds((h + k) * L, L)
                    v = wbuf[slot, j, sl] + combo_v[ty, j, sl]
                    obuf[slot, j, sl] = v
                    accs[k] = accs[k] + v
                    accs[4 + k] = accs[4 + k] + v * v
                return tuple(accs)

            s1 = jnp.sum((sums[0] + sums[1]) + (sums[2] + sums[3]))
            s2 = jnp.sum((sums[4] + sums[5]) + (sums[6] + sums[7]))
            mean = s1 * (1.0 / H)
            var = s2 * (1.0 / H) - mean * mean
            inv = _rsqrt(jnp.full((L,), var + 1e-12, jnp.float32))
            mean_v = jnp.full((L,), mean, jnp.float32)

            @plsc.parallel_loop(0, HL, step=4, unroll=2)
            def norm(h):
                for k in range(4):
                    sl = pl.ds((h + k) * L, L)
                    v = obuf[slot, j, sl]
                    o = ((v - mean_v) * inv) * gam_v[pl.ds((h + k) * L, L)] \
                        + bet_v[pl.ds((h + k) * L, L)]
                    obuf[slot, j, sl] = o
            return 0
        lax.fori_loop(0, P, j_body, 0)

        @pl.when(b + 2 < B)
        def _():
            base = (b + 2) * S + p0
            pltpu.async_copy(ids_hbm.at[pl.ds(base, P)], ids_v.at[slot],
                             ids_sem.at[slot])
            pltpu.async_copy(tt_hbm.at[pl.ds(base, P)], tt_v.at[slot],
                             tts_sem.at[slot])

        pltpu.async_copy(obuf.at[slot], out_hbm.at[b, pl.ds(p0, P)],
                         out_sem.at[slot])
        return 0
    lax.fori_loop(0, B, b_body, 0)

    for tail in (B - 2, B - 1):
        pltpu.make_async_copy(
            obuf.at[tail % 2], out_hbm.at[tail, pl.ds(p0, P)],
            out_sem.at[tail % 2]).wait()


@functools.partial(jax.jit, static_argnames=())
def kernel(input_ids, token_type_ids, word_emb, pos_emb, type_emb, gamma,
           beta):
    mesh = plsc.VectorSubcoreMesh(
        core_axis_name="c", subcore_axis_name="s",
        num_cores=NC, num_subcores=NS)
    fn = pl.kernel(
        _sc_body,
        out_type=jax.ShapeDtypeStruct((B, S, H), jnp.float32),
        mesh=mesh,
        compiler_params=pltpu.CompilerParams(needs_layout_passes=False),
        scratch_types=[
            pltpu.VMEM((2, P), jnp.int32),
            pltpu.VMEM((2, P), jnp.int32),
            pltpu.VMEM((2, P, H), jnp.float32),
            pltpu.VMEM((H,), jnp.float32),
            pltpu.VMEM((H,), jnp.float32),
            pltpu.VMEM((2, P, H), jnp.float32),
            pltpu.VMEM((2, P, H), jnp.float32),
            pltpu.SemaphoreType.DMA((2,)),
            pltpu.SemaphoreType.DMA((2,)),
            pltpu.SemaphoreType.DMA((2,)),
            pltpu.SemaphoreType.DMA((2,)),
        ],
    )
    return fn(input_ids.reshape(B * S), token_type_ids.reshape(B * S),
              word_emb, pos_emb, type_emb, gamma, beta)


# speedup vs baseline: 5.7867x; 1.1749x over previous
import functools

import jax
import jax.numpy as jnp
from jax import lax
from jax.experimental import pallas as pl
from jax.experimental.pallas import tpu as pltpu
from jax.experimental.pallas import tpu_sc as plsc

B = 128
S = 512
H = 768
L = 16
NC = 2
NS = 16
NW = NC * NS
P = S // NW
HL = H // L


def _rsqrt(x_v):
    i = lax.bitcast_convert_type(x_v, jnp.int32)
    i = jnp.full((L,), 0x5F3759DF, jnp.int32) - lax.shift_right_logical(
        i, jnp.full((L,), 1, jnp.int32))
    y = lax.bitcast_convert_type(i, jnp.float32)
    half = x_v * 0.5
    for _ in range(4):
        y = y * (1.5 - half * y * y)
    return y


NB = 4


def _sc_body(ids_hbm, tt_hbm, word_hbm, pos_hbm, type_hbm, gamma_hbm,
             beta_hbm, out_hbm, ids_v, tt_v, combo_v, gam_v, bet_v,
             buf, in_sem, out_sem, ids_sem, tts_sem):
    wid = lax.axis_index("s") * NC + lax.axis_index("c")
    p0 = wid * P

    pltpu.sync_copy(gamma_hbm, gam_v)
    pltpu.sync_copy(beta_hbm, bet_v)
    pltpu.sync_copy(pos_hbm.at[pl.ds(p0, P)], combo_v.at[0])
    pltpu.sync_copy(pos_hbm.at[pl.ds(p0, P)], combo_v.at[1])
    pltpu.sync_copy(type_hbm, buf.at[0, pl.ds(0, 2)])

    @plsc.parallel_loop(0, 2 * P * HL, step=1, unroll=4)
    def combo_body(i):
        ty = i // (P * HL)
        r = i % (P * HL)
        j = r // HL
        h = r % HL
        combo_v[ty, j, pl.ds(h * L, L)] = (
            combo_v[ty, j, pl.ds(h * L, L)] + buf[0, ty, pl.ds(h * L, L)])

    lanes = lax.iota(jnp.int32, L)

    pltpu.sync_copy(ids_hbm.at[pl.ds(p0, P)], ids_v.at[0])
    pltpu.sync_copy(tt_hbm.at[pl.ds(p0, P)], tt_v.at[0])
    pltpu.async_copy(word_hbm.at[ids_v[0, :]], buf.at[0], in_sem.at[0])
    for pb in (1, 2):
        pltpu.async_copy(ids_hbm.at[pl.ds(pb * S + p0, P)], ids_v.at[pb],
                         ids_sem.at[pb])
        pltpu.async_copy(tt_hbm.at[pl.ds(pb * S + p0, P)], tt_v.at[pb],
                         tts_sem.at[pb])

    def b_body(b, _):
        slot = lax.rem(b, NB)
        nslot = lax.rem(b + 1, NB)

        @pl.when(b + 1 < B)
        def _():
            pltpu.make_async_copy(
                ids_hbm.at[pl.ds(p0, P)], ids_v.at[nslot],
                ids_sem.at[nslot]).wait()
            pltpu.make_async_copy(
                tt_hbm.at[pl.ds(p0, P)], tt_v.at[nslot],
                tts_sem.at[nslot]).wait()

            @pl.when(b + 1 >= NB)
            def _():
                pltpu.make_async_copy(
                    buf.at[nslot], out_hbm.at[b, pl.ds(p0, P)],
                    out_sem.at[nslot]).wait()

            pltpu.async_copy(word_hbm.at[ids_v[nslot, :]], buf.at[nslot],
                             in_sem.at[nslot])

        @pl.when(b + 3 < B)
        def _():
            base = (b + 3) * S + p0
            fslot = lax.rem(b + 3, NB)
            pltpu.async_copy(ids_hbm.at[pl.ds(base, P)], ids_v.at[fslot],
                             ids_sem.at[fslot])
            pltpu.async_copy(tt_hbm.at[pl.ds(base, P)], tt_v.at[fslot],
                             tts_sem.at[fslot])

        pltpu.make_async_copy(
            pos_hbm.at[pl.ds(0, P)], buf.at[slot], in_sem.at[slot]).wait()

        tt_vec = tt_v[slot, :]
        NT = 4

        def q_body(q, _):
            j0 = q * NT
            tys = [jnp.max(jnp.where(lanes == (j0 + t), tt_vec, 0))
                   for t in range(NT)]

            zero = jnp.zeros((L,), jnp.float32)

            @plsc.parallel_loop(0, HL, step=1, unroll=2,
                                carry=(zero,) * (2 * NT))
            def sums(h, c):
                accs = list(c)
                sl = pl.ds(h * L, L)
                for t in range(NT):
                    v = (buf[slot, j0 + t, sl]
                         + combo_v[tys[t], j0 + t, sl])
                    buf[slot, j0 + t, sl] = v
                    accs[t] = accs[t] + v
                    accs[NT + t] = accs[NT + t] + v * v
                return tuple(accs)

            means, invs = [], []
            for t in range(NT):
                s1 = jnp.sum(sums[t])
                s2 = jnp.sum(sums[NT + t])
                mean = s1 * (1.0 / H)
                var = s2 * (1.0 / H) - mean * mean
                invs.append(
                    _rsqrt(jnp.full((L,), var + 1e-12, jnp.float32)))
                means.append(jnp.full((L,), mean, jnp.float32))

            @plsc.parallel_loop(0, HL, step=1, unroll=2)
            def norm(h):
                sl = pl.ds(h * L, L)
                g = gam_v[sl]
                be = bet_v[sl]
                for t in range(NT):
                    v = buf[slot, j0 + t, sl]
                    o = ((v - means[t]) * invs[t]) * g + be
                    buf[slot, j0 + t, sl] = o
            return 0
        lax.fori_loop(0, P // NT, q_body, 0)

        pltpu.async_copy(buf.at[slot], out_hbm.at[b, pl.ds(p0, P)],
                         out_sem.at[slot])
        return 0
    lax.fori_loop(0, B, b_body, 0)

    for tail in range(B - NB, B):
        pltpu.make_async_copy(
            buf.at[tail % NB], out_hbm.at[tail, pl.ds(p0, P)],
            out_sem.at[tail % NB]).wait()


@functools.partial(jax.jit, static_argnames=())
def kernel(input_ids, token_type_ids, word_emb, pos_emb, type_emb, gamma,
           beta):
    mesh = plsc.VectorSubcoreMesh(
        core_axis_name="c", subcore_axis_name="s",
        num_cores=NC, num_subcores=NS)
    fn = pl.kernel(
        _sc_body,
        out_type=jax.ShapeDtypeStruct((B, S, H), jnp.float32),
        mesh=mesh,
        compiler_params=pltpu.CompilerParams(needs_layout_passes=False),
        scratch_types=[
            pltpu.VMEM((NB, P), jnp.int32),
            pltpu.VMEM((NB, P), jnp.int32),
            pltpu.VMEM((2, P, H), jnp.float32),
            pltpu.VMEM((H,), jnp.float32),
            pltpu.VMEM((H,), jnp.float32),
            pltpu.VMEM((NB, P, H), jnp.float32),
            pltpu.SemaphoreType.DMA((NB,)),
            pltpu.SemaphoreType.DMA((NB,)),
            pltpu.SemaphoreType.DMA((NB,)),
            pltpu.SemaphoreType.DMA((NB,)),
        ],
    )
    return fn(input_ids.reshape(B * S), token_type_ids.reshape(B * S),
              word_emb, pos_emb, type_emb, gamma, beta)


# speedup vs baseline: 5.7897x; 1.0005x over previous
import functools

import jax
import jax.numpy as jnp
from jax import lax
from jax.experimental import pallas as pl
from jax.experimental.pallas import tpu as pltpu
from jax.experimental.pallas import tpu_sc as plsc

B = 128
S = 512
H = 768
L = 16
NC = 2
NS = 16
NW = NC * NS
P = S // NW
HL = H // L


def _rsqrt(x_v):
    i = lax.bitcast_convert_type(x_v, jnp.int32)
    i = jnp.full((L,), 0x5F3759DF, jnp.int32) - lax.shift_right_logical(
        i, jnp.full((L,), 1, jnp.int32))
    y = lax.bitcast_convert_type(i, jnp.float32)
    half = x_v * 0.5
    for _ in range(4):
        y = y * (1.5 - half * y * y)
    return y


NB = 6


def _sc_body(ids_hbm, tt_hbm, word_hbm, pos_hbm, type_hbm, gamma_hbm,
             beta_hbm, out_hbm, ids_v, tt_v, combo_v, gam_v, bet_v,
             buf, in_sem, out_sem, ids_sem, tts_sem):
    wid = lax.axis_index("s") * NC + lax.axis_index("c")
    p0 = wid * P

    pltpu.sync_copy(gamma_hbm, gam_v)
    pltpu.sync_copy(beta_hbm, bet_v)
    pltpu.sync_copy(pos_hbm.at[pl.ds(p0, P)], combo_v.at[0])
    pltpu.sync_copy(pos_hbm.at[pl.ds(p0, P)], combo_v.at[1])
    pltpu.sync_copy(type_hbm, buf.at[0, pl.ds(0, 2)])

    @plsc.parallel_loop(0, 2 * P * HL, step=1, unroll=4)
    def combo_body(i):
        ty = i // (P * HL)
        r = i % (P * HL)
        j = r // HL
        h = r % HL
        combo_v[ty, j, pl.ds(h * L, L)] = (
            combo_v[ty, j, pl.ds(h * L, L)] + buf[0, ty, pl.ds(h * L, L)])

    lanes = lax.iota(jnp.int32, L)

    pltpu.sync_copy(ids_hbm.at[pl.ds(p0, P)], ids_v.at[0])
    pltpu.sync_copy(tt_hbm.at[pl.ds(p0, P)], tt_v.at[0])
    pltpu.async_copy(word_hbm.at[ids_v[0, :]], buf.at[0], in_sem.at[0])
    for pb in (1, 2):
        pltpu.async_copy(ids_hbm.at[pl.ds(pb * S + p0, P)], ids_v.at[pb],
                         ids_sem.at[pb])
        pltpu.async_copy(tt_hbm.at[pl.ds(pb * S + p0, P)], tt_v.at[pb],
                         tts_sem.at[pb])

    def b_body(b, _):
        slot = lax.rem(b, NB)
        nslot = lax.rem(b + 1, NB)

        @pl.when(b + 1 < B)
        def _():
            pltpu.make_async_copy(
                ids_hbm.at[pl.ds(p0, P)], ids_v.at[nslot],
                ids_sem.at[nslot]).wait()
            pltpu.make_async_copy(
                tt_hbm.at[pl.ds(p0, P)], tt_v.at[nslot],
                tts_sem.at[nslot]).wait()

            @pl.when(b + 1 >= NB)
            def _():
                pltpu.make_async_copy(
                    buf.at[nslot], out_hbm.at[b, pl.ds(p0, P)],
                    out_sem.at[nslot]).wait()

            pltpu.async_copy(word_hbm.at[ids_v[nslot, :]], buf.at[nslot],
                             in_sem.at[nslot])

        @pl.when(b + 3 < B)
        def _():
            base = (b + 3) * S + p0
            fslot = lax.rem(b + 3, NB)
            pltpu.async_copy(ids_hbm.at[pl.ds(base, P)], ids_v.at[fslot],
                             ids_sem.at[fslot])
            pltpu.async_copy(tt_hbm.at[pl.ds(base, P)], tt_v.at[fslot],
                             tts_sem.at[fslot])

        pltpu.make_async_copy(
            pos_hbm.at[pl.ds(0, P)], buf.at[slot], in_sem.at[slot]).wait()

        tt_vec = tt_v[slot, :]
        NT = 4

        def q_body(q, _):
            j0 = q * NT
            tys = [jnp.max(jnp.where(lanes == (j0 + t), tt_vec, 0))
                   for t in range(NT)]

            zero = jnp.zeros((L,), jnp.float32)

            @plsc.parallel_loop(0, HL, step=1, unroll=2,
                                carry=(zero,) * (2 * NT))
            def sums(h, c):
                accs = list(c)
                sl = pl.ds(h * L, L)
                for t in range(NT):
                    v = (buf[slot, j0 + t, sl]
                         + combo_v[tys[t], j0 + t, sl])
                    buf[slot, j0 + t, sl] = v
                    accs[t] = accs[t] + v
                    accs[NT + t] = accs[NT + t] + v * v
                return tuple(accs)

            means, invs = [], []
            for t in range(NT):
                s1 = jnp.sum(sums[t])
                s2 = jnp.sum(sums[NT + t])
                mean = s1 * (1.0 / H)
                var = s2 * (1.0 / H) - mean * mean
                invs.append(
                    _rsqrt(jnp.full((L,), var + 1e-12, jnp.float32)))
                means.append(jnp.full((L,), mean, jnp.float32))

            @plsc.parallel_loop(0, HL, step=1, unroll=2)
            def norm(h):
                sl = pl.ds(h * L, L)
                g = gam_v[sl]
                be = bet_v[sl]
                for t in range(NT):
                    v = buf[slot, j0 + t, sl]
                    o = ((v - means[t]) * invs[t]) * g + be
                    buf[slot, j0 + t, sl] = o
            return 0
        lax.fori_loop(0, P // NT, q_body, 0)

        pltpu.async_copy(buf.at[slot], out_hbm.at[b, pl.ds(p0, P)],
                         out_sem.at[slot])
        return 0
    lax.fori_loop(0, B, b_body, 0)

    for tail in range(B - NB, B):
        pltpu.make_async_copy(
            buf.at[tail % NB], out_hbm.at[tail, pl.ds(p0, P)],
            out_sem.at[tail % NB]).wait()


@functools.partial(jax.jit, static_argnames=())
def kernel(input_ids, token_type_ids, word_emb, pos_emb, type_emb, gamma,
           beta):
    mesh = plsc.VectorSubcoreMesh(
        core_axis_name="c", subcore_axis_name="s",
        num_cores=NC, num_subcores=NS)
    fn = pl.kernel(
        _sc_body,
        out_type=jax.ShapeDtypeStruct((B, S, H), jnp.float32),
        mesh=mesh,
        compiler_params=pltpu.CompilerParams(needs_layout_passes=False),
        scratch_types=[
            pltpu.VMEM((NB, P), jnp.int32),
            pltpu.VMEM((NB, P), jnp.int32),
            pltpu.VMEM((2, P, H), jnp.float32),
            pltpu.VMEM((H,), jnp.float32),
            pltpu.VMEM((H,), jnp.float32),
            pltpu.VMEM((NB, P, H), jnp.float32),
            pltpu.SemaphoreType.DMA((NB,)),
            pltpu.SemaphoreType.DMA((NB,)),
            pltpu.SemaphoreType.DMA((NB,)),
            pltpu.SemaphoreType.DMA((NB,)),
        ],
    )
    return fn(input_ids.reshape(B * S), token_type_ids.reshape(B * S),
              word_emb, pos_emb, type_emb, gamma, beta)


# speedup vs baseline: 7.4848x; 1.2928x over previous
import functools

import jax
import jax.numpy as jnp
from jax import lax
from jax.experimental import pallas as pl
from jax.experimental.pallas import tpu as pltpu
from jax.experimental.pallas import tpu_sc as plsc

B = 128
S = 512
H = 768
L = 16
NC = 2
NS = 16
NW = NC * NS
P = S // NW
HL = H // L


def _rsqrt(x_v):
    i = lax.bitcast_convert_type(x_v, jnp.int32)
    i = jnp.full((L,), 0x5F3759DF, jnp.int32) - lax.shift_right_logical(
        i, jnp.full((L,), 1, jnp.int32))
    y = lax.bitcast_convert_type(i, jnp.float32)
    half = x_v * 0.5
    for _ in range(3):
        y = y * (1.5 - half * y * y)
    return y


NB = 6


def _sc_body(ids_hbm, tt_hbm, word_hbm, pos_hbm, type_hbm, gamma_hbm,
             beta_hbm, out_hbm, ids_v, tt_v, combo_v, gam_v, bet_v,
             buf, in_sem, out_sem, ids_sem, tts_sem):
    wid = lax.axis_index("s") * NC + lax.axis_index("c")
    p0 = wid * P

    pltpu.sync_copy(gamma_hbm, gam_v)
    pltpu.sync_copy(beta_hbm, bet_v)
    pltpu.sync_copy(pos_hbm.at[pl.ds(p0, P)], combo_v.at[0])
    pltpu.sync_copy(pos_hbm.at[pl.ds(p0, P)], combo_v.at[1])
    pltpu.sync_copy(type_hbm, buf.at[0, pl.ds(0, 2)])

    @plsc.parallel_loop(0, 2 * P * HL, step=1, unroll=4)
    def combo_body(i):
        ty = i // (P * HL)
        r = i % (P * HL)
        j = r // HL
        h = r % HL
        combo_v[ty, j, pl.ds(h * L, L)] = (
            combo_v[ty, j, pl.ds(h * L, L)] + buf[0, ty, pl.ds(h * L, L)])

    lanes = lax.iota(jnp.int32, L)

    pltpu.sync_copy(ids_hbm.at[pl.ds(p0, P)], ids_v.at[0])
    pltpu.sync_copy(tt_hbm.at[pl.ds(p0, P)], tt_v.at[0])
    pltpu.async_copy(word_hbm.at[ids_v[0, :]], buf.at[0], in_sem.at[0])
    for pb in (1, 2):
        pltpu.async_copy(ids_hbm.at[pl.ds(pb * S + p0, P)], ids_v.at[pb],
                         ids_sem.at[pb])
        pltpu.async_copy(tt_hbm.at[pl.ds(pb * S + p0, P)], tt_v.at[pb],
                         tts_sem.at[pb])

    def b_body(b, _):
        slot = lax.rem(b, NB)
        nslot = lax.rem(b + 1, NB)

        @pl.when(b + 1 < B)
        def _():
            pltpu.make_async_copy(
                ids_hbm.at[pl.ds(p0, P)], ids_v.at[nslot],
                ids_sem.at[nslot]).wait()
            pltpu.make_async_copy(
                tt_hbm.at[pl.ds(p0, P)], tt_v.at[nslot],
                tts_sem.at[nslot]).wait()

            @pl.when(b + 1 >= NB)
            def _():
                pltpu.make_async_copy(
                    buf.at[nslot], out_hbm.at[b, pl.ds(p0, P)],
                    out_sem.at[nslot]).wait()

            pltpu.async_copy(word_hbm.at[ids_v[nslot, :]], buf.at[nslot],
                             in_sem.at[nslot])

        @pl.when(b + 3 < B)
        def _():
            base = (b + 3) * S + p0
            fslot = lax.rem(b + 3, NB)
            pltpu.async_copy(ids_hbm.at[pl.ds(base, P)], ids_v.at[fslot],
                             ids_sem.at[fslot])
            pltpu.async_copy(tt_hbm.at[pl.ds(base, P)], tt_v.at[fslot],
                             tts_sem.at[fslot])

        pltpu.make_async_copy(
            pos_hbm.at[pl.ds(0, P)], buf.at[slot], in_sem.at[slot]).wait()

        tt_vec = tt_v[slot, :]
        NT = 8

        def q_body(q, _):
            j0 = q * NT
            tys = [jnp.max(jnp.where(lanes == (j0 + t), tt_vec, 0))
                   for t in range(NT)]

            zero = jnp.zeros((L,), jnp.float32)

            @plsc.parallel_loop(0, HL, step=1, unroll=1,
                                carry=(zero,) * (2 * NT))
            def sums(h, c):
                accs = list(c)
                sl = pl.ds(h * L, L)
                for t in range(NT):
                    v = (buf[slot, j0 + t, sl]
                         + combo_v[tys[t], j0 + t, sl])
                    buf[slot, j0 + t, sl] = v
                    accs[t] = accs[t] + v
                    accs[NT + t] = accs[NT + t] + v * v
                return tuple(accs)

            means, invs = [], []
            for t in range(NT):
                s1 = jnp.sum(sums[t])
                s2 = jnp.sum(sums[NT + t])
                mean = s1 * (1.0 / H)
                var = s2 * (1.0 / H) - mean * mean
                invs.append(
                    _rsqrt(jnp.full((L,), var + 1e-12, jnp.float32)))
                means.append(jnp.full((L,), mean, jnp.float32))

            @plsc.parallel_loop(0, HL, step=1, unroll=1)
            def norm(h):
                sl = pl.ds(h * L, L)
                g = gam_v[sl]
                be = bet_v[sl]
                for t in range(NT):
                    v = buf[slot, j0 + t, sl]
                    o = ((v - means[t]) * invs[t]) * g + be
                    buf[slot, j0 + t, sl] = o
            return 0
        lax.fori_loop(0, P // NT, q_body, 0)

        pltpu.async_copy(buf.at[slot], out_hbm.at[b, pl.ds(p0, P)],
                         out_sem.at[slot])
        return 0
    lax.fori_loop(0, B, b_body, 0)

    for tail in range(B - NB, B):
        pltpu.make_async_copy(
            buf.at[tail % NB], out_hbm.at[tail, pl.ds(p0, P)],
            out_sem.at[tail % NB]).wait()


@functools.partial(jax.jit, static_argnames=())
def kernel(input_ids, token_type_ids, word_emb, pos_emb, type_emb, gamma,
           beta):
    mesh = plsc.VectorSubcoreMesh(
        core_axis_name="c", subcore_axis_name="s",
        num_cores=NC, num_subcores=NS)
    fn = pl.kernel(
        _sc_body,
        out_type=jax.ShapeDtypeStruct((B, S, H), jnp.float32),
        mesh=mesh,
        compiler_params=pltpu.CompilerParams(needs_layout_passes=False),
        scratch_types=[
            pltpu.VMEM((NB, P), jnp.int32),
            pltpu.VMEM((NB, P), jnp.int32),
            pltpu.VMEM((2, P, H), jnp.float32),
            pltpu.VMEM((H,), jnp.float32),
            pltpu.VMEM((H,), jnp.float32),
            pltpu.VMEM((NB, P, H), jnp.float32),
            pltpu.SemaphoreType.DMA((NB,)),
            pltpu.SemaphoreType.DMA((NB,)),
            pltpu.SemaphoreType.DMA((NB,)),
            pltpu.SemaphoreType.DMA((NB,)),
        ],
    )
    return fn(input_ids.reshape(B * S), token_type_ids.reshape(B * S),
              word_emb, pos_emb, type_emb, gamma, beta)
